# baseline (speedup 1.0000x reference)
import functools

import jax
import jax.numpy as jnp
from jax import lax
from jax.experimental import pallas as pl
from jax.experimental.pallas import tpu as pltpu
from jax.experimental.pallas import tpu_sc as plsc

T, D, FF, E, DOUT, TOP_K_ = 2048, 2048, 4096, 8, 9, 2
LP = 128
RB = 256
NRB = T // RB
BT = 128
NB = T // BT
NT = NB + E - 1
FFB = 2048
NFF = FF // FFB
NEG_PAD = -1.0e30
NEG_MASK = -3.0e38
_XCH = 16
_NCH = (T // 32) // _XCH


def _route_body(x_ref, gw_ref, gb_ref, pos_ref, off_ref, loss_ref,
                oh_ref, r_ref, acc_ref):
    s = pl.program_id(0)

    @pl.when(s == 0)
    def _():
        acc_ref[...] = jnp.zeros_like(acc_ref)

    @pl.when(s < NRB)
    def _():
        x = x_ref[...]
        logits = jnp.dot(x, gw_ref[...],
                         preferred_element_type=jnp.float32) + gb_ref[...]
        lane = lax.broadcasted_iota(jnp.int32, (RB, LP), 1).astype(jnp.float32)
        m1 = jnp.max(logits, axis=1, keepdims=True)
        idx1 = jnp.min(jnp.where(logits == m1, lane, 1e9), axis=1,
                       keepdims=True)
        logits2 = jnp.where(lane == idx1, NEG_MASK, logits)
        m2 = jnp.max(logits2, axis=1, keepdims=True)
        idx2 = jnp.min(jnp.where(logits2 == m2, lane, 1e9), axis=1,
                       keepdims=True)
        estar = jnp.maximum(idx1, idx2)
        oh = (lane == estar).astype(jnp.float32)

        ri = lax.broadcasted_iota(jnp.int32, (RB, RB), 0)
        ci = lax.broadcasted_iota(jnp.int32, (RB, RB), 1)
        ltri = (ri >= ci).astype(jnp.float32)
        csum = jnp.dot(ltri, oh, preferred_element_type=jnp.float32)
        csum = csum + acc_ref[0:1, :]
        oh_ref[pl.ds(s * RB, RB), :] = oh
        r_ref[pl.ds(s * RB, RB), :] = oh * (csum - 1.0)

        se = jnp.exp(logits - m1)
        ssum = jnp.sum(se, axis=1, keepdims=True)
        lse = m1 + jnp.log(ssum)
        probs = se / ssum
        oh1 = (lane == idx1).astype(jnp.float32)
        acc_ref[0:1, :] = acc_ref[0:1, :] + jnp.sum(oh, axis=0, keepdims=True)
        acc_ref[1:2, :] = acc_ref[1:2, :] + jnp.sum(lse * lse)
        acc_ref[2:3, :] = acc_ref[2:3, :] + jnp.sum(probs, axis=0,
                                                    keepdims=True)
        acc_ref[3:4, :] = acc_ref[3:4, :] + jnp.sum(oh1, axis=0,
                                                    keepdims=True)

    @pl.when(s == NRB)
    def _():
        cnt = acc_ref[0:1, :]
        li = lax.broadcasted_iota(jnp.int32, (LP, LP), 0)
        lj = lax.broadcasted_iota(jnp.int32, (LP, LP), 1)
        stri = (li < lj).astype(jnp.float32)
        offe = jnp.dot(cnt, stri, preferred_element_type=jnp.float32,
                       precision=jax.lax.Precision.HIGHEST)
        posl = r_ref[...] + oh_ref[...] * offe
        pos = jnp.sum(posl, axis=1, keepdims=True)
        pos_ref[...] = jnp.broadcast_to(pos, (T, LP)).astype(jnp.int32)
        off_ref[...] = jnp.broadcast_to(offe, (8, LP)).astype(jnp.int32)
        lz = acc_ref[1:2, :] * jnp.float32(1.0 / T)
        lb = jnp.sum(acc_ref[3:4, :] * acc_ref[2:3, :], axis=1,
                     keepdims=True) * jnp.float32(0.01 * E / (T * float(T)))
        loss_ref[...] = jnp.concatenate(
            [lz, jnp.broadcast_to(lb, (1, LP)),
             jnp.zeros((6, LP), jnp.float32)], axis=0)


_route_call = pl.pallas_call(
    _route_body,
    grid=(NRB + 1,),
    in_specs=[
        pl.BlockSpec((RB, D), lambda s: (jnp.minimum(s, NRB - 1), 0)),
        pl.BlockSpec((D, LP), lambda s: (0, 0)),
        pl.BlockSpec((1, LP), lambda s: (0, 0)),
    ],
    out_specs=[
        pl.BlockSpec((T, LP), lambda s: (0, 0)),
        pl.BlockSpec((8, LP), lambda s: (0, 0)),
        pl.BlockSpec((8, LP), lambda s: (0, 0)),
    ],
    out_shape=[
        jax.ShapeDtypeStruct((T, LP), jnp.int32),
        jax.ShapeDtypeStruct((8, LP), jnp.int32),
        jax.ShapeDtypeStruct((8, LP), jnp.float32),
    ],
    scratch_shapes=[
        pltpu.VMEM((T, LP), jnp.float32),
        pltpu.VMEM((T, LP), jnp.float32),
        pltpu.VMEM((8, LP), jnp.float32),
    ],
    compiler_params=pltpu.CompilerParams(
        dimension_semantics=("arbitrary",)),
)


DEPTH = 2
NSTEP = NFF * NT


def _ffn_body(sched_ref, ftab_ref, xs_ref, w1_any, w2_ref, b1_ref, b2_ref,
              out_ref, ring_ref, sems):
    j = pl.program_id(0)
    i = pl.program_id(1)
    k = j * NT + i

    def _fetch(f, slot):
        e_f = ftab_ref[0, f]
        j_f = ftab_ref[1, f]
        return pltpu.make_async_copy(
            w1_any.at[e_f, :, pl.ds(j_f * FFB, FFB)],
            ring_ref.at[slot], sems.at[slot])

    @pl.when(k == 0)
    def _():
        out_ref[...] = jnp.zeros_like(out_ref)
        for d in range(DEPTH):
            _fetch(d, d).start()

    iss = sched_ref[6, k]

    @pl.when(iss >= 0)
    def _():
        _fetch(iss, sched_ref[7, k]).start()

    slot = sched_ref[4, k]

    @pl.when(sched_ref[5, k] == 1)
    def _():
        _fetch(sched_ref[8, k], slot).wait()

    lo = sched_ref[2, i]
    hi = sched_ref[3, i]

    @pl.when(lo < hi)
    def _():
        b = sched_ref[1, i]
        x = xs_ref[pl.ds(b * BT, BT), :].astype(jnp.bfloat16)
        w1b = ring_ref[slot].astype(jnp.bfloat16)
        h = jnp.maximum(
            jnp.dot(x, w1b, preferred_element_type=jnp.float32)
            + b1_ref[0, 0], 0.0)
        part = jnp.dot(h.astype(jnp.bfloat16),
                       w2_ref[0].astype(jnp.bfloat16),
                       preferred_element_type=jnp.float32)
        b2row = jnp.where(j == 0, 1.0, 0.0) * b2_ref[0]
        part = part + b2row
        row = b * BT + lax.broadcasted_iota(jnp.int32, (BT, LP), 0)
        act = (row >= lo) & (row < hi)
        out_ref[pl.ds(b * BT, BT), :] += jnp.where(act, part, 0.0)


_ffn_call = pl.pallas_call(
    _ffn_body,
    grid_spec=pltpu.PrefetchScalarGridSpec(
        num_scalar_prefetch=2,
        grid=(NFF, NT),
        in_specs=[
            pl.BlockSpec((T, D), lambda j, i, s, f: (0, 0)),
            pl.BlockSpec(memory_space=pl.ANY),
            pl.BlockSpec((1, FFB, LP), lambda j, i, s, f: (s[0, i], j, 0)),
            pl.BlockSpec((1, 1, 1, FFB), lambda j, i, s, f: (s[0, i], j, 0, 0)),
            pl.BlockSpec((1, 1, LP), lambda j, i, s, f: (s[0, i], 0, 0)),
        ],
        out_specs=pl.BlockSpec((T, LP), lambda j, i, s, f: (0, 0)),
        scratch_shapes=[
            pltpu.VMEM((DEPTH, D, FFB), jnp.float32),
            pltpu.SemaphoreType.DMA((DEPTH,)),
        ],
    ),
    out_shape=jax.ShapeDtypeStruct((T, LP), jnp.float32),
    compiler_params=pltpu.CompilerParams(
        dimension_semantics=("arbitrary", "arbitrary"),
        vmem_limit_bytes=100 * 1024 * 1024),
)


@functools.lru_cache(maxsize=None)
def _sc_kernels():
    mesh = plsc.VectorSubcoreMesh(core_axis_name="c", subcore_axis_name="s")

    gx = functools.partial(
        pl.kernel, mesh=mesh,
        out_type=jax.ShapeDtypeStruct((T, D), jnp.float32),
        scratch_types=[
            pltpu.VMEM((_XCH,), jnp.int32),
            pltpu.VMEM((_XCH,), jnp.int32),
            pltpu.VMEM((_XCH, D), jnp.float32),
            pltpu.VMEM((_XCH, D), jnp.float32),
            pltpu.SemaphoreType.DMA,
            pltpu.SemaphoreType.DMA,
        ])(_scatter_x_body)
    go = functools.partial(
        pl.kernel, mesh=mesh,
        out_type=jax.ShapeDtypeStruct((T, LP), jnp.float32),
        scratch_types=[
            pltpu.VMEM((T // 32,), jnp.int32),
            pltpu.VMEM((T // 32, LP), jnp.float32),
            pltpu.SemaphoreType.DMA,
        ])(_gather_out_body)
    return gx, go


def _scatter_x_body(x_hbm, pos_hbm, xs_hbm, idx16a, idx16b,
                    buf0, buf1, sem0, sem1):
    c = lax.axis_index("c")
    s = lax.axis_index("s")
    wid = s * 2 + c
    npw = T // 32
    base = wid * npw
    bufs = (buf0, buf1)
    idxs = (idx16a, idx16b)
    sems = (sem0, sem1)
    for ch in range(_NCH):
        pltpu.sync_copy(pos_hbm.at[pl.ds(base + ch * _XCH, _XCH)],
                        idxs[ch % 2])
        pltpu.async_copy(x_hbm.at[pl.ds(base + ch * _XCH, _XCH)],
                         bufs[ch % 2], sems[ch % 2]).wait()
        pltpu.async_copy(bufs[ch % 2], xs_hbm.at[idxs[ch % 2]],
                         sems[ch % 2]).wait()


def _gather_out_body(osort_hbm, pos_hbm, res_hbm, idx_v, buf, sem):
    c = lax.axis_index("c")
    s = lax.axis_index("s")
    wid = s * 2 + c
    npw = T // 32
    base = wid * npw
    pltpu.sync_copy(pos_hbm.at[pl.ds(base, npw)], idx_v)
    pltpu.async_copy(osort_hbm.at[idx_v], buf, sem).wait()
    pltpu.sync_copy(buf, res_hbm.at[pl.ds(base, npw)])


def kernel(inputs, gate_W, gate_b, W1, b1, W2, b2):
    inputs = inputs.astype(jnp.float32)
    gwp = jnp.pad(gate_W.astype(jnp.float32), ((0, 0), (0, LP - E)))
    gbp = jnp.pad(gate_b.astype(jnp.float32), (0, LP - E),
                  constant_values=NEG_PAD).reshape(1, LP)

    pos2d, off2d, loss2d = _route_call(inputs, gwp, gbp)
    pos = pos2d[:, 0]
    off9 = jnp.concatenate(
        [off2d[0, :E], jnp.array([T], jnp.int32)])

    rows_first = jnp.arange(NB, dtype=jnp.int32) * BT
    e_first = (jnp.searchsorted(off9, rows_first, side="right") - 1
               ).astype(jnp.int32)
    e_last = (jnp.searchsorted(off9, rows_first + (BT - 1), side="right") - 1
              ).astype(jnp.int32)
    npairs = e_last - e_first + 1
    cum = jnp.concatenate([jnp.zeros(1, jnp.int32),
                           jnp.cumsum(npairs)]).astype(jnp.int32)
    tidx = jnp.arange(NT, dtype=jnp.int32)
    b_i = jnp.clip(jnp.searchsorted(cum, tidx, side="right") - 1,
                   0, NB - 1).astype(jnp.int32)
    e_i = jnp.clip(e_first[b_i] + (tidx - cum[b_i]), 0, E - 1)
    valid = tidx < cum[NB]
    lo = jnp.where(valid, jnp.maximum(off9[e_i], b_i * BT), 0)
    hi = jnp.where(valid, jnp.minimum(off9[e_i + 1], (b_i + 1) * BT), 0)

    m0 = jnp.concatenate([jnp.ones(1, jnp.bool_), e_i[1:] != e_i[:-1]])
    run_id = jnp.cumsum(m0.astype(jnp.int32)) - 1
    m = run_id[NT - 1] + 1
    nftot = NFF * m
    run_e = jnp.zeros(NT, jnp.int32).at[run_id].set(e_i)
    farange = jnp.arange(NSTEP, dtype=jnp.int32)
    fe = run_e[jnp.clip(farange % jnp.maximum(m, 1), 0, NT - 1)]
    jf = jnp.clip(farange // jnp.maximum(m, 1), 0, NFF - 1)
    ftab = jnp.stack([fe, jf]).astype(jnp.int32)

    kk = farange
    ii = kk % NT
    fi = (kk // NT) * m + run_id[ii]
    first = jnp.concatenate(
        [jnp.ones(1, jnp.int32), (fi[1:] != fi[:-1]).astype(jnp.int32)])
    issf = jnp.where((kk > 0) & (first == 1) & (fi + DEPTH - 1 < nftot),
                     fi + DEPTH - 1, -1)
    isslot = jnp.where(issf >= 0, issf % DEPTH, 0)
    sched = (jnp.zeros((9, NSTEP), jnp.int32)
             .at[0, :NT].set(e_i).at[1, :NT].set(b_i)
             .at[2, :NT].set(lo).at[3, :NT].set(hi)
             .at[4].set(fi % DEPTH).at[5].set(first)
             .at[6].set(issf).at[7].set(isslot).at[8].set(fi))

    scatter_x, gather_out = _sc_kernels()
    xs = scatter_x(inputs, pos)

    w2p = jnp.pad(W2.astype(jnp.float32), ((0, 0), (0, 0), (0, LP - DOUT)))
    b2p = jnp.pad(b2.astype(jnp.float32),
                  ((0, 0), (0, LP - DOUT))).reshape(E, 1, LP)
    b1r = b1.astype(jnp.float32).reshape(E, NFF, 1, FFB)
    osort = _ffn_call(sched, ftab, xs, W1.astype(jnp.float32), w2p, b1r, b2p)

    res = gather_out(osort, pos)
    results = res[:, :DOUT]
    return results, loss2d[0, 0], loss2d[1, 0]


# speedup vs baseline: 1.0733x; 1.0733x over previous
import functools

import jax
import jax.numpy as jnp
from jax import lax
from jax.experimental import pallas as pl
from jax.experimental.pallas import tpu as pltpu
from jax.experimental.pallas import tpu_sc as plsc

T, D, FF, E, DOUT, TOP_K_ = 2048, 2048, 4096, 8, 9, 2
LP = 128
RB = 256
NRB = T // RB
BT = 128
NB = T // BT
NT = NB + E
FFB = 2048
NFF = FF // FFB
NEG_PAD = -1.0e30
NEG_MASK = -3.0e38
_XCH = 16
_NCH = (T // 32) // _XCH


def _route_body(x_ref, gw_ref, gb_ref, pos_ref, off_ref, loss_ref,
                oh_ref, r_ref, acc_ref):
    s = pl.program_id(0)

    @pl.when(s == 0)
    def _():
        acc_ref[...] = jnp.zeros_like(acc_ref)

    @pl.when(s < NRB)
    def _():
        x = x_ref[...]
        logits = jnp.dot(x, gw_ref[...],
                         preferred_element_type=jnp.float32) + gb_ref[...]
        lane = lax.broadcasted_iota(jnp.int32, (RB, LP), 1).astype(jnp.float32)
        m1 = jnp.max(logits, axis=1, keepdims=True)
        idx1 = jnp.min(jnp.where(logits == m1, lane, 1e9), axis=1,
                       keepdims=True)
        logits2 = jnp.where(lane == idx1, NEG_MASK, logits)
        m2 = jnp.max(logits2, axis=1, keepdims=True)
        idx2 = jnp.min(jnp.where(logits2 == m2, lane, 1e9), axis=1,
                       keepdims=True)
        estar = jnp.maximum(idx1, idx2)
        oh = (lane == estar).astype(jnp.float32)

        ri = lax.broadcasted_iota(jnp.int32, (RB, RB), 0)
        ci = lax.broadcasted_iota(jnp.int32, (RB, RB), 1)
        ltri = (ri >= ci).astype(jnp.float32)
        csum = jnp.dot(ltri, oh, preferred_element_type=jnp.float32)
        csum = csum + acc_ref[0:1, :]
        oh_ref[pl.ds(s * RB, RB), :] = oh
        r_ref[pl.ds(s * RB, RB), :] = oh * (csum - 1.0)

        se = jnp.exp(logits - m1)
        ssum = jnp.sum(se, axis=1, keepdims=True)
        lse = m1 + jnp.log(ssum)
        probs = se / ssum
        oh1 = (lane == idx1).astype(jnp.float32)
        acc_ref[0:1, :] = acc_ref[0:1, :] + jnp.sum(oh, axis=0, keepdims=True)
        acc_ref[1:2, :] = acc_ref[1:2, :] + jnp.sum(lse * lse)
        acc_ref[2:3, :] = acc_ref[2:3, :] + jnp.sum(probs, axis=0,
                                                    keepdims=True)
        acc_ref[3:4, :] = acc_ref[3:4, :] + jnp.sum(oh1, axis=0,
                                                    keepdims=True)

    @pl.when(s == NRB)
    def _():
        cnt = acc_ref[0:1, :]
        li = lax.broadcasted_iota(jnp.int32, (LP, LP), 0)
        lj = lax.broadcasted_iota(jnp.int32, (LP, LP), 1)
        stri = (li < lj).astype(jnp.float32)
        offe = jnp.dot(cnt, stri, preferred_element_type=jnp.float32,
                       precision=jax.lax.Precision.HIGHEST)
        posl = r_ref[...] + oh_ref[...] * offe
        pos = jnp.sum(posl, axis=1, keepdims=True)
        pos_ref[...] = jnp.broadcast_to(pos, (T, LP)).astype(jnp.int32)
        off_ref[...] = jnp.broadcast_to(offe, (8, LP)).astype(jnp.int32)
        lz = acc_ref[1:2, :] * jnp.float32(1.0 / T)
        lb = jnp.sum(acc_ref[3:4, :] * acc_ref[2:3, :], axis=1,
                     keepdims=True) * jnp.float32(0.01 * E / (T * float(T)))
        loss_ref[...] = jnp.concatenate(
            [lz, jnp.broadcast_to(lb, (1, LP)),
             jnp.zeros((6, LP), jnp.float32)], axis=0)


_route_call = pl.pallas_call(
    _route_body,
    grid=(NRB + 1,),
    in_specs=[
        pl.BlockSpec((RB, D), lambda s: (jnp.minimum(s, NRB - 1), 0)),
        pl.BlockSpec((D, LP), lambda s: (0, 0)),
        pl.BlockSpec((1, LP), lambda s: (0, 0)),
    ],
    out_specs=[
        pl.BlockSpec((T, LP), lambda s: (0, 0)),
        pl.BlockSpec((8, LP), lambda s: (0, 0)),
        pl.BlockSpec((8, LP), lambda s: (0, 0)),
    ],
    out_shape=[
        jax.ShapeDtypeStruct((T, LP), jnp.int32),
        jax.ShapeDtypeStruct((8, LP), jnp.int32),
        jax.ShapeDtypeStruct((8, LP), jnp.float32),
    ],
    scratch_shapes=[
        pltpu.VMEM((T, LP), jnp.float32),
        pltpu.VMEM((T, LP), jnp.float32),
        pltpu.VMEM((8, LP), jnp.float32),
    ],
    compiler_params=pltpu.CompilerParams(
        dimension_semantics=("arbitrary",)),
)


DEPTH = 2
NSTEP = NFF * NT


def _ffn_body(sched_ref, ftab_ref, xs_ref, w1_any, w2_ref, b1_ref, b2_ref,
              out_ref, ring_ref, sems):
    j = pl.program_id(0)
    i = pl.program_id(1)
    k = j * NT + i

    def _fetch(f, slot):
        e_f = ftab_ref[0, f]
        j_f = ftab_ref[1, f]
        return pltpu.make_async_copy(
            w1_any.at[e_f, :, pl.ds(j_f * FFB, FFB)],
            ring_ref.at[slot], sems.at[slot])

    @pl.when(k == 0)
    def _():
        out_ref[...] = jnp.zeros_like(out_ref)
        for d in range(DEPTH):
            _fetch(d, d).start()

    iss = sched_ref[6, k]

    @pl.when(iss >= 0)
    def _():
        _fetch(iss, sched_ref[7, k]).start()

    slot = sched_ref[4, k]

    @pl.when(sched_ref[5, k] == 1)
    def _():
        _fetch(sched_ref[8, k], slot).wait()

    lo = sched_ref[2, i]
    hi = sched_ref[3, i]

    @pl.when(lo < hi)
    def _():
        st = pl.multiple_of(sched_ref[1, i], 8)
        x = xs_ref[pl.ds(st, BT), :]
        w1b = ring_ref[slot]
        h = jnp.maximum(
            jnp.dot(x, w1b, preferred_element_type=jnp.float32)
            + b1_ref[0, 0], 0.0)
        part = jnp.dot(h, w2_ref[0], preferred_element_type=jnp.float32)
        b2row = jnp.where(j == 0, 1.0, 0.0) * b2_ref[0]
        part = part + b2row
        row = st + lax.broadcasted_iota(jnp.int32, (BT, LP), 0)
        act = (row >= lo) & (row < hi)
        out_ref[pl.ds(st, BT), :] += jnp.where(act, part, 0.0)


_ffn_call = pl.pallas_call(
    _ffn_body,
    grid_spec=pltpu.PrefetchScalarGridSpec(
        num_scalar_prefetch=2,
        grid=(NFF, NT),
        in_specs=[
            pl.BlockSpec((T, D), lambda j, i, s, f: (0, 0)),
            pl.BlockSpec(memory_space=pl.ANY),
            pl.BlockSpec((1, FFB, LP), lambda j, i, s, f: (s[0, i], j, 0)),
            pl.BlockSpec((1, 1, 1, FFB), lambda j, i, s, f: (s[0, i], j, 0, 0)),
            pl.BlockSpec((1, 1, LP), lambda j, i, s, f: (s[0, i], 0, 0)),
        ],
        out_specs=pl.BlockSpec((T, LP), lambda j, i, s, f: (0, 0)),
        scratch_shapes=[
            pltpu.VMEM((DEPTH, D, FFB), jnp.float32),
            pltpu.SemaphoreType.DMA((DEPTH,)),
        ],
    ),
    out_shape=jax.ShapeDtypeStruct((T, LP), jnp.float32),
    compiler_params=pltpu.CompilerParams(
        dimension_semantics=("arbitrary", "arbitrary"),
        vmem_limit_bytes=100 * 1024 * 1024),
)


@functools.lru_cache(maxsize=None)
def _sc_kernels():
    mesh = plsc.VectorSubcoreMesh(core_axis_name="c", subcore_axis_name="s")

    gx = functools.partial(
        pl.kernel, mesh=mesh,
        out_type=jax.ShapeDtypeStruct((T, D), jnp.float32),
        scratch_types=[
            pltpu.VMEM((_XCH,), jnp.int32),
            pltpu.VMEM((_XCH,), jnp.int32),
            pltpu.VMEM((_XCH, D), jnp.float32),
            pltpu.VMEM((_XCH, D), jnp.float32),
            pltpu.SemaphoreType.DMA,
            pltpu.SemaphoreType.DMA,
        ])(_scatter_x_body)
    go = functools.partial(
        pl.kernel, mesh=mesh,
        out_type=jax.ShapeDtypeStruct((T, LP), jnp.float32),
        scratch_types=[
            pltpu.VMEM((T // 32,), jnp.int32),
            pltpu.VMEM((T // 32, LP), jnp.float32),
            pltpu.SemaphoreType.DMA,
        ])(_gather_out_body)
    return gx, go


def _scatter_x_body(x_hbm, pos_hbm, xs_hbm, idx16a, idx16b,
                    buf0, buf1, sem0, sem1):
    c = lax.axis_index("c")
    s = lax.axis_index("s")
    wid = s * 2 + c
    npw = T // 32
    base = wid * npw
    bufs = (buf0, buf1)
    idxs = (idx16a, idx16b)
    sems = (sem0, sem1)
    for ch in range(_NCH):
        pltpu.sync_copy(pos_hbm.at[pl.ds(base + ch * _XCH, _XCH)],
                        idxs[ch % 2])
        pltpu.async_copy(x_hbm.at[pl.ds(base + ch * _XCH, _XCH)],
                         bufs[ch % 2], sems[ch % 2]).wait()
        pltpu.async_copy(bufs[ch % 2], xs_hbm.at[idxs[ch % 2]],
                         sems[ch % 2]).wait()


def _gather_out_body(osort_hbm, pos_hbm, res_hbm, idx_v, buf, sem):
    c = lax.axis_index("c")
    s = lax.axis_index("s")
    wid = s * 2 + c
    npw = T // 32
    base = wid * npw
    pltpu.sync_copy(pos_hbm.at[pl.ds(base, npw)], idx_v)
    pltpu.async_copy(osort_hbm.at[idx_v], buf, sem).wait()
    pltpu.sync_copy(buf, res_hbm.at[pl.ds(base, npw)])


def kernel(inputs, gate_W, gate_b, W1, b1, W2, b2):
    inputs = inputs.astype(jnp.float32)
    gwp = jnp.pad(gate_W.astype(jnp.float32), ((0, 0), (0, LP - E)))
    gbp = jnp.pad(gate_b.astype(jnp.float32), (0, LP - E),
                  constant_values=NEG_PAD).reshape(1, LP)

    pos2d, off2d, loss2d = _route_call(inputs, gwp, gbp)
    pos = pos2d[:, 0]
    off9 = jnp.concatenate(
        [off2d[0, :E], jnp.array([T], jnp.int32)])

    cnts = off9[1:] - off9[:E]
    start8 = (off9[:E] // 8) * 8
    nt_e = jnp.where(cnts > 0, (off9[1:] - start8 + BT - 1) // BT, 0)
    cumt = jnp.concatenate([jnp.zeros(1, jnp.int32),
                            jnp.cumsum(nt_e)]).astype(jnp.int32)
    tidx = jnp.arange(NT, dtype=jnp.int32)
    e_i = jnp.clip(jnp.searchsorted(cumt, tidx, side="right") - 1,
                   0, E - 1).astype(jnp.int32)
    k_e = tidx - cumt[e_i]
    start_u = start8[e_i] + k_e * BT
    valid = tidx < cumt[E]
    lo = jnp.where(valid, jnp.maximum(off9[e_i], start_u), 0)
    hi = jnp.where(valid, jnp.minimum(off9[e_i + 1], start_u + BT), 0)
    b_i = jnp.where(valid, jnp.clip(start_u, 0, T - BT), 0)

    m0 = jnp.concatenate([jnp.ones(1, jnp.bool_), e_i[1:] != e_i[:-1]])
    run_id = jnp.cumsum(m0.astype(jnp.int32)) - 1
    m = run_id[NT - 1] + 1
    nftot = NFF * m
    run_e = jnp.zeros(NT, jnp.int32).at[run_id].set(e_i)
    farange = jnp.arange(NSTEP, dtype=jnp.int32)
    fe = run_e[jnp.clip(farange % jnp.maximum(m, 1), 0, NT - 1)]
    jf = jnp.clip(farange // jnp.maximum(m, 1), 0, NFF - 1)
    ftab = jnp.stack([fe, jf]).astype(jnp.int32)

    kk = farange
    ii = kk % NT
    fi = (kk // NT) * m + run_id[ii]
    first = jnp.concatenate(
        [jnp.ones(1, jnp.int32), (fi[1:] != fi[:-1]).astype(jnp.int32)])
    issf = jnp.where((kk > 0) & (first == 1) & (fi + DEPTH - 1 < nftot),
                     fi + DEPTH - 1, -1)
    isslot = jnp.where(issf >= 0, issf % DEPTH, 0)
    sched = (jnp.zeros((9, NSTEP), jnp.int32)
             .at[0, :NT].set(e_i).at[1, :NT].set(b_i)
             .at[2, :NT].set(lo).at[3, :NT].set(hi)
             .at[4].set(fi % DEPTH).at[5].set(first)
             .at[6].set(issf).at[7].set(isslot).at[8].set(fi))

    scatter_x, gather_out = _sc_kernels()
    xs = scatter_x(inputs, pos)

    w2p = jnp.pad(W2.astype(jnp.float32), ((0, 0), (0, 0), (0, LP - DOUT)))
    b2p = jnp.pad(b2.astype(jnp.float32),
                  ((0, 0), (0, LP - DOUT))).reshape(E, 1, LP)
    b1r = b1.astype(jnp.float32).reshape(E, NFF, 1, FFB)
    osort = _ffn_call(sched, ftab, xs, W1.astype(jnp.float32), w2p, b1r, b2p)

    res = gather_out(osort, pos)
    results = res[:, :DOUT]
    return results, loss2d[0, 0], loss2d[1, 0]
